# baseline (speedup 1.0000x reference)
import jax
import jax.numpy as jnp
from jax.experimental import pallas as pl
from jax.experimental.pallas import tpu as pltpu

_B = 256
_T = 64
_DN = 2000
_PN = 1500
_AD = 32
_BB = 8


def _branch(x_ref, lens, tgt, w1_ref, b1_ref, w2_ref, b2, out_ref, d):
    x = x_ref[...]
    x2 = x.reshape(_BB * _T, d)
    h = jnp.tanh(
        jax.lax.dot_general(x2, w1_ref[...], (((1,), (0,)), ((), ())),
                            preferred_element_type=jnp.float32)
        + b1_ref[...])
    s = jax.lax.dot_general(h, w2_ref[...], (((1,), (0,)), ((), ())),
                            preferred_element_type=jnp.float32)
    s = s.reshape(_BB, _T) + b2
    t_ids = jax.lax.broadcasted_iota(jnp.int32, (_BB, _T), 1)
    s = jnp.where(t_ids < lens, s, -1e9)
    m = jnp.max(s, axis=1, keepdims=True)
    e = jnp.exp(s - m)
    p = e / jnp.sum(e, axis=1, keepdims=True)
    col = jax.lax.broadcasted_iota(jnp.int32, (_BB, _T, d), 2)
    hit = col == tgt.reshape(_BB, 1, 1)
    out_ref[...] = jnp.minimum(x + jnp.where(hit, p[:, :, None], 0.0), 1.0)


def _fused_kernel(idx_hbm, wd1_hbm, bd1_hbm, wd2_hbm, bd2_hbm,
                  wp1_hbm, bp1_hbm, wp2_hbm, bp2_hbm,
                  dx_ref, px_ref, dout_ref, pout_ref,
                  idx_v, wd1_v, bd1_v, wd2_v, bd2_v,
                  wp1_v, bp1_v, wp2_v, bp2_v, sem):
    i = pl.program_id(0)

    @pl.when(i == 0)
    def _prefetch():
        for src, dst in ((idx_hbm, idx_v), (wd1_hbm, wd1_v),
                         (bd1_hbm, bd1_v), (wd2_hbm, wd2_v),
                         (bd2_hbm, bd2_v), (wp1_hbm, wp1_v),
                         (bp1_hbm, bp1_v), (wp2_hbm, wp2_v),
                         (bp2_hbm, bp2_v)):
            copy = pltpu.make_async_copy(src, dst, sem)
            copy.start()
            copy.wait()

    rows = idx_v[pl.ds(i * _BB, _BB), :]
    lens = rows[:, 0:1]
    tgtd = rows[:, 1:2]
    tgtp = rows[:, 2:3]
    _branch(dx_ref, lens, tgtd, wd1_v, bd1_v, wd2_v, bd2_v[0, 0],
            dout_ref, _DN)
    _branch(px_ref, lens, tgtp, wp1_v, bp1_v, wp2_v, bp2_v[0, 0],
            pout_ref, _PN)


@jax.jit
def kernel(diagnosis_x, procedure_x, lens, target_diagnoses,
           target_procedures, Wd1, bd1, Wd2, bd2, Wp1, bp1, Wp2, bp2):
    idx = jnp.stack([lens.astype(jnp.int32),
                     target_diagnoses.astype(jnp.int32),
                     target_procedures.astype(jnp.int32),
                     jnp.zeros((_B,), jnp.int32)], axis=1)
    bd1r = bd1.reshape(1, _AD)
    bp1r = bp1.reshape(1, _AD)
    bd2r = bd2.reshape(1, 1)
    bp2r = bp2.reshape(1, 1)

    _any = pl.BlockSpec(memory_space=pl.ANY)
    grid = (_B // _BB,)
    dout, pout = pl.pallas_call(
        _fused_kernel,
        grid=grid,
        in_specs=[
            _any, _any, _any, _any, _any, _any, _any, _any, _any,
            pl.BlockSpec((_BB, _T, _DN), lambda i: (i, 0, 0)),
            pl.BlockSpec((_BB, _T, _PN), lambda i: (i, 0, 0)),
        ],
        out_specs=[
            pl.BlockSpec((_BB, _T, _DN), lambda i: (i, 0, 0)),
            pl.BlockSpec((_BB, _T, _PN), lambda i: (i, 0, 0)),
        ],
        out_shape=[
            jax.ShapeDtypeStruct((_B, _T, _DN), jnp.float32),
            jax.ShapeDtypeStruct((_B, _T, _PN), jnp.float32),
        ],
        scratch_shapes=[
            pltpu.VMEM((_B, 4), jnp.int32),
            pltpu.VMEM((_DN, _AD), jnp.float32),
            pltpu.VMEM((1, _AD), jnp.float32),
            pltpu.VMEM((_AD, 1), jnp.float32),
            pltpu.VMEM((1, 1), jnp.float32),
            pltpu.VMEM((_PN, _AD), jnp.float32),
            pltpu.VMEM((1, _AD), jnp.float32),
            pltpu.VMEM((_AD, 1), jnp.float32),
            pltpu.VMEM((1, 1), jnp.float32),
            pltpu.SemaphoreType.DMA,
        ],
    )(idx, Wd1, bd1r, Wd2, bd2r, Wp1, bp1r, Wp2, bp2r,
      diagnosis_x, procedure_x)
    return dout, pout
